# Initial kernel scaffold; baseline (speedup 1.0000x reference)
import jax
import jax.numpy as jnp
from jax.experimental import pallas as pl


def kernel(x, skip, edge_index, kid_trans, kid_up, kid1, kid2, kid3, W_trans, W_up, W1, W2, W3, g_tbn, b_tbn, g1, b1, g2, b2, g3, b3):
    raise NotImplementedError("write your pallas kernel here")


# speedup vs baseline: 3.2813x; 3.2813x over previous
import functools

import jax
import jax.numpy as jnp
from jax import lax
from jax.experimental import pallas as pl
from jax.experimental.pallas import tpu as pltpu
from jax.experimental.pallas import tpu_sc as plsc

N = 10000
E = 160000
C = 128
K = 27
EPS = 1e-5
SLOPE = 0.01

NC = 2
NS = 16
NT = NC * NS
EPT = E // NT
CHUNK = 125
NCHUNK = EPT // CHUNK
NPAD = 10240
RPT = NPAD // NS

RB = 400
NB = N // RB


def _prep_body(src_ref, kt_ref, ku_ref, f1_ref, f2_ref):
    src = src_ref[...]
    f1_ref[...] = kt_ref[...] * N + src
    f2_ref[...] = ku_ref[...] * N + src


def _prep(src2d, kt2d, ku2d):
    return pl.pallas_call(
        _prep_body,
        out_shape=(
            jax.ShapeDtypeStruct((E // C, C), jnp.int32),
            jax.ShapeDtypeStruct((E // C, C), jnp.int32),
        ),
    )(src2d, kt2d, ku2d)


def _mm_body(x_ref, w_ref, o_ref):
    xb = x_ref[...]
    for k in range(K):
        o_ref[k] = jnp.dot(xb, w_ref[k], preferred_element_type=jnp.float32)


def _mm(feat, W):
    return pl.pallas_call(
        _mm_body,
        grid=(NB,),
        in_specs=[
            pl.BlockSpec((RB, C), lambda i: (i, 0)),
            pl.BlockSpec((K, C, C), lambda i: (0, 0, 0)),
        ],
        out_specs=pl.BlockSpec((K, RB, C), lambda i: (0, i, 0)),
        out_shape=jax.ShapeDtypeStruct((K, N, C), jnp.float32),
    )(feat, W)


def _sc_body(y_hbm, flat_hbm, dst_hbm, zeros_hbm, out_hbm,
             flat_v, dst_v, rows_v, acc_sh, sem):
    cid = lax.axis_index("c")
    sid = lax.axis_index("s")
    t = cid * NS + sid
    pltpu.sync_copy(flat_hbm.at[t], flat_v)
    pltpu.sync_copy(dst_hbm.at[t], dst_v)
    rows = pl.ds(sid * RPT, RPT)
    pltpu.sync_copy(zeros_hbm.at[rows], acc_sh.at[rows])
    plsc.subcore_barrier()

    def body(j, carry):
        pltpu.async_copy(y_hbm.at[flat_v.at[j]], rows_v, sem).wait()
        pltpu.sync_copy(rows_v, acc_sh.at[dst_v.at[j]], add=True)
        return carry

    lax.fori_loop(0, NCHUNK, body, 0)
    plsc.subcore_barrier()
    pltpu.sync_copy(acc_sh.at[rows], out_hbm.at[cid, rows])


def _sc_segsum(y_flat, flat_idx, dst_idx, zeros):
    mesh = plsc.VectorSubcoreMesh(
        core_axis_name="c", subcore_axis_name="s",
        num_cores=NC, num_subcores=NS)
    f = functools.partial(
        pl.kernel,
        out_type=jax.ShapeDtypeStruct((NC, NPAD, C), jnp.float32),
        mesh=mesh,
        scratch_types=[
            pltpu.VMEM((NCHUNK, CHUNK), jnp.int32),
            pltpu.VMEM((NCHUNK, CHUNK), jnp.int32),
            pltpu.VMEM((CHUNK, C), jnp.float32),
            pltpu.VMEM_SHARED((NPAD, C), jnp.float32),
            pltpu.SemaphoreType.DMA,
        ],
    )(_sc_body)
    return f(y_flat, flat_idx, dst_idx, zeros)


def _bn_mid_body(p_ref, g_ref, b_ref, o_ref):
    h = p_ref[0] + p_ref[1]
    h = jnp.where(h >= 0, h, SLOPE * h)
    m = jnp.mean(h, axis=0, keepdims=True)
    d = h - m
    v = jnp.mean(d * d, axis=0, keepdims=True)
    o_ref[...] = g_ref[...] * d / jnp.sqrt(v + EPS) + b_ref[...]


def _bn_mid(p, g, b):
    return pl.pallas_call(
        _bn_mid_body,
        out_shape=jax.ShapeDtypeStruct((N, C), jnp.float32),
    )(p, g.reshape(1, C), b.reshape(1, C))


def _bn_fin_body(p_ref, s_ref, g_ref, b_ref, o_ref):
    h = p_ref[0] + p_ref[1] + s_ref[...]
    m = jnp.mean(h, axis=0, keepdims=True)
    d = h - m
    v = jnp.mean(d * d, axis=0, keepdims=True)
    o_ref[...] = g_ref[...] * d / jnp.sqrt(v + EPS) + b_ref[...]


def _bn_fin(p, skip, g, b):
    return pl.pallas_call(
        _bn_fin_body,
        out_shape=jax.ShapeDtypeStruct((N, C), jnp.float32),
    )(p, skip, g.reshape(1, C), b.reshape(1, C))


def kernel(x, skip, edge_index, kid_trans, kid_up, kid1, kid2, kid3,
           W_trans, W_up, W1, W2, W3,
           g_tbn, b_tbn, g1, b1, g2, b2, g3, b3):
    src2d = edge_index[0].reshape(E // C, C)
    flat1, flat2 = _prep(src2d,
                         kid_trans.reshape(E // C, C),
                         kid_up.reshape(E // C, C))
    flat1 = flat1.reshape(NT, NCHUNK, CHUNK)
    flat2 = flat2.reshape(NT, NCHUNK, CHUNK)
    dst3d = edge_index[1].reshape(NT, NCHUNK, CHUNK)
    zeros = jnp.zeros((NPAD, C), jnp.float32)

    y1 = _mm(x, W_trans).reshape(K * N, C)
    p1 = _sc_segsum(y1, flat1, dst3d, zeros)
    h = _bn_mid(p1[:, :N], g_tbn, b_tbn)

    y2 = _mm(h, W_up).reshape(K * N, C)
    p2 = _sc_segsum(y2, flat2, dst3d, zeros)
    return _bn_fin(p2[:, :N], skip, g3, b3)
